# baseline (speedup 1.0000x reference)
import functools

import jax
import jax.numpy as jnp
import numpy as np
from jax import lax
from jax.experimental import pallas as pl
from jax.experimental.pallas import tpu as pltpu
from jax.experimental.pallas import tpu_sc as plsc

B = 64
MAXLEN = 2048
D_MODEL = 768
VOCAB = 5

NC = 2
NS = 16
LANES = 16
NW = NC * NS

TOKENS = B * MAXLEN
PASSES = 4
L_TILE = LANES
SLICE_ROWS = L_TILE * VOCAB
D_VREGS = D_MODEL // LANES


def _pe_scaled() -> np.ndarray:
    pos = np.arange(MAXLEN)[:, np.newaxis]
    i = np.arange(D_MODEL)[np.newaxis, :]
    angle_rates = 1.0 / np.power(10000, 2 * (i // 2) / np.float32(D_MODEL))
    angle_rads = pos * angle_rates
    angle_rads[:, 0::2] = np.sin(angle_rads[:, 0::2])
    angle_rads[:, 1::2] = np.cos(angle_rads[:, 1::2])
    return (angle_rads / np.sqrt(np.float32(D_MODEL))).astype(np.float32)


_PE_SCALED = _pe_scaled()


def _fuse_body(table_ref, pe_ref, out_ref):
    out_ref[...] = pe_ref[...][:, None, :] + table_ref[...][None, :, :]


_L_BLK = 256


def _build_fused(table):
    pe = jnp.asarray(_PE_SCALED)
    return pl.pallas_call(
        _fuse_body,
        grid=(MAXLEN // _L_BLK,),
        in_specs=[
            pl.BlockSpec((VOCAB, D_MODEL), lambda i: (0, 0)),
            pl.BlockSpec((_L_BLK, D_MODEL), lambda i: (i, 0)),
        ],
        out_specs=pl.BlockSpec((_L_BLK, VOCAB, D_MODEL), lambda i: (i, 0, 0)),
        out_shape=jax.ShapeDtypeStruct((MAXLEN, VOCAB, D_MODEL), jnp.float32),
    )(table, pe)


@functools.partial(
    pl.kernel,
    out_type=jax.ShapeDtypeStruct((TOKENS * D_MODEL,), jnp.float32),
    mesh=plsc.VectorSubcoreMesh(core_axis_name="c", subcore_axis_name="s"),
    scratch_types=[
        pltpu.VMEM((LANES,), jnp.int32),
        pltpu.VMEM((LANES,), jnp.int32),
        pltpu.VMEM((SLICE_ROWS * D_MODEL,), jnp.float32),
        pltpu.VMEM((L_TILE * D_MODEL,), jnp.float32),
        pltpu.VMEM((L_TILE * D_MODEL,), jnp.float32),
        pltpu.SemaphoreType.DMA,
        pltpu.SemaphoreType.DMA,
        pltpu.SemaphoreType.DMA,
        pltpu.SemaphoreType.DMA,
    ],
)
def _sc_embed(tok_hbm, fused_hbm, out_hbm,
              tokb0, tokb1, slice_v, stage0, stage1,
              tsem0, tsem1, osem0, osem1):
    cid = lax.axis_index("c")
    sid = lax.axis_index("s")
    wid = sid * NC + cid

    def l0_of(p):
        return p * (MAXLEN // PASSES) + wid * L_TILE

    def tok_ref(p, b):
        return tok_hbm.at[pl.ds(b * MAXLEN + l0_of(p), L_TILE)]

    def out_ref(p, b):
        return out_hbm.at[
            pl.ds((b * MAXLEN + l0_of(p)) * D_MODEL, L_TILE * D_MODEL)]

    def build_block(tokb, st):
        tok16 = tokb[...]
        for l in range(L_TILE):
            start = (l * VOCAB + tok16[l]) * D_MODEL
            for k in range(D_VREGS):
                st[pl.ds(l * D_MODEL + k * LANES, LANES)] = slice_v[
                    pl.ds(start + k * LANES, LANES)]

    def run_pass(p, carry):
        pltpu.sync_copy(
            fused_hbm.at[pl.ds(l0_of(p) * VOCAB * D_MODEL,
                               SLICE_ROWS * D_MODEL)],
            slice_v)
        pltpu.async_copy(tok_ref(p, 0), tokb0, tsem0)

        def pair(bb, carry2):
            b_even = 2 * bb
            pltpu.make_async_copy(tok_ref(p, b_even), tokb0, tsem0).wait()
            pltpu.async_copy(tok_ref(p, b_even + 1), tokb1, tsem1)

            @pl.when(bb > 0)
            def _():
                pltpu.make_async_copy(stage0, out_ref(p, b_even - 2),
                                      osem0).wait()

            build_block(tokb0, stage0)
            pltpu.async_copy(stage0, out_ref(p, b_even), osem0)

            pltpu.make_async_copy(tok_ref(p, b_even + 1), tokb1, tsem1).wait()

            @pl.when(bb + 1 < B // 2)
            def _():
                pltpu.async_copy(tok_ref(p, b_even + 2), tokb0, tsem0)

            @pl.when(bb > 0)
            def _():
                pltpu.make_async_copy(stage1, out_ref(p, b_even - 1),
                                      osem1).wait()

            build_block(tokb1, stage1)
            pltpu.async_copy(stage1, out_ref(p, b_even + 1), osem1)
            return carry2

        lax.fori_loop(0, B // 2, pair, 0)
        pltpu.make_async_copy(stage0, out_ref(p, B - 2), osem0).wait()
        pltpu.make_async_copy(stage1, out_ref(p, B - 1), osem1).wait()
        return carry

    lax.fori_loop(0, PASSES, run_pass, 0)


@jax.jit
def kernel(rnatok, table):
    fused = _build_fused(table)
    fused_flat = fused.reshape(MAXLEN * VOCAB * D_MODEL)
    out = _sc_embed(rnatok.reshape(TOKENS), fused_flat)
    return out.reshape(B, MAXLEN, D_MODEL)


# speedup vs baseline: 1.7300x; 1.7300x over previous
import functools

import jax
import jax.numpy as jnp
import numpy as np
from jax import lax
from jax.experimental import pallas as pl
from jax.experimental.pallas import tpu as pltpu
from jax.experimental.pallas import tpu_sc as plsc

B = 64
MAXLEN = 2048
D_MODEL = 768
VOCAB = 5

NC = 2
NS = 16
LANES = 16
NW = NC * NS

TOKENS = B * MAXLEN
PASSES = 4
L_TILE = LANES
SLICE_ROWS = L_TILE * VOCAB
D_VREGS = D_MODEL // LANES


def _pe_scaled() -> np.ndarray:
    pos = np.arange(MAXLEN)[:, np.newaxis]
    i = np.arange(D_MODEL)[np.newaxis, :]
    angle_rates = 1.0 / np.power(10000, 2 * (i // 2) / np.float32(D_MODEL))
    angle_rads = pos * angle_rates
    angle_rads[:, 0::2] = np.sin(angle_rads[:, 0::2])
    angle_rads[:, 1::2] = np.cos(angle_rads[:, 1::2])
    return (angle_rads / np.sqrt(np.float32(D_MODEL))).astype(np.float32)


_PE_SCALED = _pe_scaled()


def _fuse_body(table_ref, pe_ref, out_ref):
    out_ref[...] = pe_ref[...][:, None, :] + table_ref[...][None, :, :]


_L_BLK = 256


def _build_fused(table):
    pe = jnp.asarray(_PE_SCALED)
    return pl.pallas_call(
        _fuse_body,
        grid=(MAXLEN // _L_BLK,),
        in_specs=[
            pl.BlockSpec((VOCAB, D_MODEL), lambda i: (0, 0)),
            pl.BlockSpec((_L_BLK, D_MODEL), lambda i: (i, 0)),
        ],
        out_specs=pl.BlockSpec((_L_BLK, VOCAB, D_MODEL), lambda i: (i, 0, 0)),
        out_shape=jax.ShapeDtypeStruct((MAXLEN, VOCAB, D_MODEL), jnp.float32),
    )(table, pe)


@functools.partial(
    pl.kernel,
    out_type=jax.ShapeDtypeStruct((TOKENS * D_MODEL,), jnp.float32),
    mesh=plsc.VectorSubcoreMesh(core_axis_name="c", subcore_axis_name="s"),
    scratch_types=[
        pltpu.VMEM((LANES,), jnp.int32),
        pltpu.VMEM((LANES,), jnp.int32),
        pltpu.VMEM((SLICE_ROWS * D_MODEL,), jnp.float32),
        pltpu.VMEM((L_TILE * D_MODEL,), jnp.float32),
        pltpu.VMEM((L_TILE * D_MODEL,), jnp.float32),
        pltpu.SemaphoreType.DMA,
        pltpu.SemaphoreType.DMA,
        pltpu.SemaphoreType.DMA,
        pltpu.SemaphoreType.DMA,
    ],
)
def _sc_embed(tok_hbm, fused_hbm, out_hbm,
              tokb0, tokb1, slice_v, stage0, stage1,
              tsem0, tsem1, osem0, osem1):
    cid = lax.axis_index("c")
    sid = lax.axis_index("s")
    wid = sid * NC + cid

    def l0_of(p):
        return p * (MAXLEN // PASSES) + wid * L_TILE

    def tok_ref(p, b):
        return tok_hbm.at[pl.ds(b * MAXLEN + l0_of(p), L_TILE)]

    def out_ref(p, b):
        return out_hbm.at[
            pl.ds((b * MAXLEN + l0_of(p)) * D_MODEL, L_TILE * D_MODEL)]

    def build_block(tokb, st):
        LAT = 8
        tok16 = tokb[...]
        starts = [(l * VOCAB + tok16[l]) * D_MODEL for l in range(L_TILE)]
        n = L_TILE * D_VREGS
        pending = []
        for i in range(n + LAT):
            if i < n:
                l, k = divmod(i, D_VREGS)
                pending.append(slice_v[pl.ds(starts[l] + k * LANES, LANES)])
            if i >= LAT:
                j = i - LAT
                st[pl.ds(j * LANES, LANES)] = pending[j]

    def run_pass(p, carry):
        pltpu.sync_copy(
            fused_hbm.at[pl.ds(l0_of(p) * VOCAB * D_MODEL,
                               SLICE_ROWS * D_MODEL)],
            slice_v)
        pltpu.async_copy(tok_ref(p, 0), tokb0, tsem0)

        def pair(bb, carry2):
            b_even = 2 * bb
            pltpu.make_async_copy(tok_ref(p, b_even), tokb0, tsem0).wait()
            pltpu.async_copy(tok_ref(p, b_even + 1), tokb1, tsem1)

            @pl.when(bb > 0)
            def _():
                pltpu.make_async_copy(stage0, out_ref(p, b_even - 2),
                                      osem0).wait()

            build_block(tokb0, stage0)
            pltpu.async_copy(stage0, out_ref(p, b_even), osem0)

            pltpu.make_async_copy(tok_ref(p, b_even + 1), tokb1, tsem1).wait()

            @pl.when(bb + 1 < B // 2)
            def _():
                pltpu.async_copy(tok_ref(p, b_even + 2), tokb0, tsem0)

            @pl.when(bb > 0)
            def _():
                pltpu.make_async_copy(stage1, out_ref(p, b_even - 1),
                                      osem1).wait()

            build_block(tokb1, stage1)
            pltpu.async_copy(stage1, out_ref(p, b_even + 1), osem1)
            return carry2

        lax.fori_loop(0, B // 2, pair, 0)
        pltpu.make_async_copy(stage0, out_ref(p, B - 2), osem0).wait()
        pltpu.make_async_copy(stage1, out_ref(p, B - 1), osem1).wait()
        return carry

    lax.fori_loop(0, PASSES, run_pass, 0)


@jax.jit
def kernel(rnatok, table):
    fused = _build_fused(table)
    fused_flat = fused.reshape(MAXLEN * VOCAB * D_MODEL)
    out = _sc_embed(rnatok.reshape(TOKENS), fused_flat)
    return out.reshape(B, MAXLEN, D_MODEL)


# speedup vs baseline: 1.7377x; 1.0045x over previous
import functools

import jax
import jax.numpy as jnp
import numpy as np
from jax import lax
from jax.experimental import pallas as pl
from jax.experimental.pallas import tpu as pltpu
from jax.experimental.pallas import tpu_sc as plsc

B = 64
MAXLEN = 2048
D_MODEL = 768
VOCAB = 5

NC = 2
NS = 16
LANES = 16
NW = NC * NS

TOKENS = B * MAXLEN
PASSES = 4
L_TILE = LANES
SLICE_ROWS = L_TILE * VOCAB
D_VREGS = D_MODEL // LANES


def _pe_scaled() -> np.ndarray:
    pos = np.arange(MAXLEN)[:, np.newaxis]
    i = np.arange(D_MODEL)[np.newaxis, :]
    angle_rates = 1.0 / np.power(10000, 2 * (i // 2) / np.float32(D_MODEL))
    angle_rads = pos * angle_rates
    angle_rads[:, 0::2] = np.sin(angle_rads[:, 0::2])
    angle_rads[:, 1::2] = np.cos(angle_rads[:, 1::2])
    return (angle_rads / np.sqrt(np.float32(D_MODEL))).astype(np.float32)


_PE_SCALED = _pe_scaled()


def _fuse_body(table_ref, pe_ref, out_ref):
    out_ref[...] = pe_ref[...][:, None, :] + table_ref[...][None, :, :]


_L_BLK = 256


def _build_fused(table):
    pe = jnp.asarray(_PE_SCALED)
    return pl.pallas_call(
        _fuse_body,
        grid=(MAXLEN // _L_BLK,),
        in_specs=[
            pl.BlockSpec((VOCAB, D_MODEL), lambda i: (0, 0)),
            pl.BlockSpec((_L_BLK, D_MODEL), lambda i: (i, 0)),
        ],
        out_specs=pl.BlockSpec((_L_BLK, VOCAB, D_MODEL), lambda i: (i, 0, 0)),
        out_shape=jax.ShapeDtypeStruct((MAXLEN, VOCAB, D_MODEL), jnp.float32),
    )(table, pe)


@functools.partial(
    pl.kernel,
    out_type=jax.ShapeDtypeStruct((TOKENS * D_MODEL,), jnp.float32),
    mesh=plsc.VectorSubcoreMesh(core_axis_name="c", subcore_axis_name="s"),
    scratch_types=[
        pltpu.VMEM((B * L_TILE,), jnp.int32),
        pltpu.VMEM((SLICE_ROWS * D_MODEL,), jnp.float32),
        pltpu.VMEM((L_TILE * D_MODEL,), jnp.float32),
        pltpu.VMEM((L_TILE * D_MODEL,), jnp.float32),
        pltpu.SemaphoreType.DMA,
        pltpu.SemaphoreType.DMA,
        pltpu.SemaphoreType.DMA,
    ],
)
def _sc_embed(tok_hbm, fused_hbm, out_hbm,
              tokp_v, slice_v, stage0, stage1,
              tsem, osem0, osem1):
    cid = lax.axis_index("c")
    sid = lax.axis_index("s")
    wid = sid * NC + cid

    def l0_of(p):
        return p * (MAXLEN // PASSES) + wid * L_TILE

    def tok_ref(p, b):
        return tok_hbm.at[pl.ds(b * MAXLEN + l0_of(p), L_TILE)]

    def out_ref(p, b):
        return out_hbm.at[
            pl.ds((b * MAXLEN + l0_of(p)) * D_MODEL, L_TILE * D_MODEL)]

    def build_block(b, st):
        LAT = 8
        tok16 = tokp_v[pl.ds(b * L_TILE, L_TILE)]
        starts = [(l * VOCAB + tok16[l]) * D_MODEL for l in range(L_TILE)]
        n = L_TILE * D_VREGS
        pending = []
        for i in range(n + LAT):
            if i < n:
                l, k = divmod(i, D_VREGS)
                pending.append(slice_v[pl.ds(starts[l] + k * LANES, LANES)])
            if i >= LAT:
                j = i - LAT
                st[pl.ds(j * LANES, LANES)] = pending[j]

    def run_pass(p, carry):
        for b in range(B):
            pltpu.async_copy(tok_ref(p, b),
                             tokp_v.at[pl.ds(b * L_TILE, L_TILE)], tsem)
        pltpu.sync_copy(
            fused_hbm.at[pl.ds(l0_of(p) * VOCAB * D_MODEL,
                               SLICE_ROWS * D_MODEL)],
            slice_v)
        for b in range(B):
            pltpu.make_async_copy(tok_ref(p, b),
                                  tokp_v.at[pl.ds(b * L_TILE, L_TILE)],
                                  tsem).wait()

        def pair(bb, carry2):
            b_even = 2 * bb

            @pl.when(bb > 0)
            def _():
                pltpu.make_async_copy(stage0, out_ref(p, b_even - 2),
                                      osem0).wait()

            build_block(b_even, stage0)
            pltpu.async_copy(stage0, out_ref(p, b_even), osem0)

            @pl.when(bb > 0)
            def _():
                pltpu.make_async_copy(stage1, out_ref(p, b_even - 1),
                                      osem1).wait()

            build_block(b_even + 1, stage1)
            pltpu.async_copy(stage1, out_ref(p, b_even + 1), osem1)
            return carry2

        lax.fori_loop(0, B // 2, pair, 0)
        pltpu.make_async_copy(stage0, out_ref(p, B - 2), osem0).wait()
        pltpu.make_async_copy(stage1, out_ref(p, B - 1), osem1).wait()
        return carry

    lax.fori_loop(0, PASSES, run_pass, 0)


@jax.jit
def kernel(rnatok, table):
    fused = _build_fused(table)
    fused_flat = fused.reshape(MAXLEN * VOCAB * D_MODEL)
    out = _sc_embed(rnatok.reshape(TOKENS), fused_flat)
    return out.reshape(B, MAXLEN, D_MODEL)


# speedup vs baseline: 4.1803x; 2.4056x over previous
import functools

import jax
import jax.numpy as jnp
import numpy as np
from jax import lax
from jax.experimental import pallas as pl
from jax.experimental.pallas import tpu as pltpu
from jax.experimental.pallas import tpu_sc as plsc

B = 64
MAXLEN = 2048
D_MODEL = 768
VOCAB = 5

NC = 2
NS = 16
LANES = 16
NW = NC * NS

TOKENS = B * MAXLEN
T_PER_W = TOKENS // NW
CHUNK = 32
N_CHUNKS = T_PER_W // CHUNK


def _pe_scaled() -> np.ndarray:
    pos = np.arange(MAXLEN)[:, np.newaxis]
    i = np.arange(D_MODEL)[np.newaxis, :]
    angle_rates = 1.0 / np.power(10000, 2 * (i // 2) / np.float32(D_MODEL))
    angle_rads = pos * angle_rates
    angle_rads[:, 0::2] = np.sin(angle_rads[:, 0::2])
    angle_rads[:, 1::2] = np.cos(angle_rads[:, 1::2])
    return (angle_rads / np.sqrt(np.float32(D_MODEL))).astype(np.float32)


_PE_SCALED = _pe_scaled()


def _fuse_body(table_ref, pe_ref, out_ref):
    out_ref[...] = table_ref[...][:, None, :] + pe_ref[...][None, :, :]


_L_BLK = 256


def _build_fused(table):
    pe = jnp.asarray(_PE_SCALED)
    return pl.pallas_call(
        _fuse_body,
        grid=(MAXLEN // _L_BLK,),
        in_specs=[
            pl.BlockSpec((VOCAB, D_MODEL), lambda i: (0, 0)),
            pl.BlockSpec((_L_BLK, D_MODEL), lambda i: (i, 0)),
        ],
        out_specs=pl.BlockSpec((VOCAB, _L_BLK, D_MODEL), lambda i: (0, i, 0)),
        out_shape=jax.ShapeDtypeStruct((VOCAB, MAXLEN, D_MODEL), jnp.float32),
    )(table, pe)


@functools.partial(
    pl.kernel,
    out_type=jax.ShapeDtypeStruct((TOKENS, D_MODEL), jnp.float32),
    mesh=plsc.VectorSubcoreMesh(core_axis_name="c", subcore_axis_name="s"),
    scratch_types=[
        pltpu.VMEM((T_PER_W,), jnp.int32),
        pltpu.VMEM((CHUNK, D_MODEL), jnp.float32),
        pltpu.VMEM((CHUNK, D_MODEL), jnp.float32),
        pltpu.VMEM((CHUNK, D_MODEL), jnp.float32),
        pltpu.VMEM((CHUNK, D_MODEL), jnp.float32),
        pltpu.SemaphoreType.DMA,
        pltpu.SemaphoreType.DMA,
        pltpu.SemaphoreType.DMA,
        pltpu.SemaphoreType.DMA,
        pltpu.SemaphoreType.DMA,
        pltpu.SemaphoreType.DMA,
        pltpu.SemaphoreType.DMA,
        pltpu.SemaphoreType.DMA,
    ],
)
def _sc_gather(tok_hbm, fused_hbm, out_hbm, idx_all,
               rows0, rows1, rows2, rows3,
               gsem0, gsem1, gsem2, gsem3, osem0, osem1, osem2, osem3):
    wid = lax.axis_index("s") * NC + lax.axis_index("c")
    base = wid * T_PER_W

    pltpu.sync_copy(tok_hbm.at[pl.ds(base, T_PER_W)], idx_all)

    def to_idx(r, carry):
        sl = pl.ds(r * LANES, LANES)
        tok16 = idx_all[sl]
        t16 = (base + r * LANES) + lax.iota(jnp.int32, LANES)
        idx_all[sl] = (tok16 << 11) | (t16 & (MAXLEN - 1))
        return carry

    lax.fori_loop(0, T_PER_W // LANES, to_idx, 0)

    def fire_gather(c, rows_v, sem):
        pltpu.async_copy(
            fused_hbm.at[idx_all.at[pl.ds(c * CHUNK, CHUNK)]], rows_v, sem)

    def wait_gather(c, rows_v, sem):
        pltpu.make_async_copy(
            fused_hbm.at[idx_all.at[pl.ds(c * CHUNK, CHUNK)]], rows_v, sem).wait()

    def fire_out(c, rows_v, sem):
        pltpu.async_copy(rows_v, out_hbm.at[pl.ds(base + c * CHUNK, CHUNK)], sem)

    def wait_out(c, rows_v, sem):
        pltpu.make_async_copy(
            rows_v, out_hbm.at[pl.ds(base + c * CHUNK, CHUNK)], sem).wait()

    rows = (rows0, rows1, rows2, rows3)
    gsem = (gsem0, gsem1, gsem2, gsem3)
    osem = (osem0, osem1, osem2, osem3)

    fire_gather(0, rows[0], gsem[0])
    fire_gather(1, rows[1], gsem[1])

    def quad(q, carry):
        for j in range(4):
            c = 4 * q + j
            wait_gather(c, rows[j], gsem[j])

            @pl.when(c >= 2)
            def _():
                wait_out(c - 2, rows[(j + 2) % 4], osem[(j + 2) % 4])

            @pl.when(c + 2 < N_CHUNKS)
            def _():
                fire_gather(c + 2, rows[(j + 2) % 4], gsem[(j + 2) % 4])

            fire_out(c, rows[j], osem[j])
        return carry

    lax.fori_loop(0, N_CHUNKS // 4, quad, 0)
    wait_out(N_CHUNKS - 2, rows[2], osem[2])
    wait_out(N_CHUNKS - 1, rows[3], osem[3])


@jax.jit
def kernel(rnatok, table):
    fused = _build_fused(table)
    fused2d = fused.reshape(VOCAB * MAXLEN, D_MODEL)
    tok_flat = rnatok.reshape(TOKENS)
    out = _sc_gather(tok_flat, fused2d)
    return out.reshape(B, MAXLEN, D_MODEL)
